# Initial kernel scaffold; baseline (speedup 1.0000x reference)
import jax
import jax.numpy as jnp
from jax.experimental import pallas as pl


def kernel(atom_in_fea, nbr_fea, nbr_fea_idx, W1, b1, W2, b2, bn_gamma, bn_beta):
    raise NotImplementedError("write your pallas kernel here")


# speedup vs baseline: 2.0319x; 2.0319x over previous
import functools

import jax
import jax.numpy as jnp
from jax import lax
from jax.experimental import pallas as pl
from jax.experimental.pallas import tpu as pltpu
from jax.experimental.pallas import tpu_sc as plsc

N, M, F, D_E, H = 10000, 32, 128, 16, 128
BN_BLK = 200
NBLK = N // BN_BLK

def _q_kernel(atom_ref, w1b_ref, q_ref):
    q_ref[...] = jnp.dot(atom_ref[...], w1b_ref[...],
                         preferred_element_type=jnp.float32)


def _compute_q(atom, w1b):
    return pl.pallas_call(
        _q_kernel,
        grid=(NBLK,),
        in_specs=[
            pl.BlockSpec((BN_BLK, F), lambda i: (i, 0)),
            pl.BlockSpec((F, H), lambda i: (0, 0)),
        ],
        out_specs=pl.BlockSpec((BN_BLK, H), lambda i: (i, 0)),
        out_shape=jax.ShapeDtypeStruct((N, H), jnp.float32),
    )(atom, w1b)


_NC, _NS = 2, 16
_NW = _NC * _NS
_E_PER_W = (N * M) // _NW
_CHUNK = 400
_NCHUNK = _E_PER_W // _CHUNK


def _gather_body(q_hbm, idx_hbm, out_hbm, idx_v, rows_v, sem):
    wid = lax.axis_index("s") * _NC + lax.axis_index("c")
    base = wid * _E_PER_W

    def step(c, _):
        off = base + c * _CHUNK
        pltpu.sync_copy(idx_hbm.at[pl.ds(off, _CHUNK)], idx_v)
        pltpu.async_copy(q_hbm.at[idx_v], rows_v, sem).wait()
        pltpu.sync_copy(rows_v, out_hbm.at[pl.ds(off, _CHUNK)])
        return ()

    lax.fori_loop(0, _NCHUNK, step, (), unroll=False)


def _gather_rows(q, idx_flat):
    mesh = plsc.VectorSubcoreMesh(core_axis_name="c", subcore_axis_name="s")
    fn = functools.partial(
        pl.kernel, mesh=mesh,
        out_type=jax.ShapeDtypeStruct((N * M, H), jnp.float32),
        scratch_types=[
            pltpu.VMEM((_CHUNK,), jnp.int32),
            pltpu.VMEM((_CHUNK, H), jnp.float32),
            pltpu.SemaphoreType.DMA,
        ],
    )(_gather_body)
    return fn(q, idx_flat)


def _softplus(x):
    return jnp.maximum(x, 0.0) + jnp.log1p(jnp.exp(-jnp.abs(x)))


def _msg_kernel(atom_ref, qg_ref, nbr_ref, idxf_ref, w1a_ref, w1c_ref,
                b1_ref, w2_ref, b2_ref, out_ref, psum_ref, psumsq_ref):
    atom = atom_ref[...]
    p = jnp.dot(atom, w1a_ref[...],
                preferred_element_type=jnp.float32) + b1_ref[...]
    nbr2 = nbr_ref[...].reshape(BN_BLK * M, D_E)
    e2 = jnp.dot(nbr2, w1c_ref[...],
                 preferred_element_type=jnp.float32)
    a3 = qg_ref[...] + e2.reshape(BN_BLK, M, H) + p[:, None, :]
    h3 = _softplus(a3)
    m3 = idxf_ref[...] != 0.0
    hsum = jnp.sum(jnp.where(m3, h3, 0.0), axis=1)
    cnt = jnp.sum(jnp.where(m3, 1.0, 0.0), axis=1)
    msg = (jnp.dot(hsum, w2_ref[...], preferred_element_type=jnp.float32)
           + cnt * b2_ref[...])
    out_pre = atom + msg
    out_ref[...] = out_pre
    psum_ref[...] = jnp.sum(out_pre, axis=0, keepdims=True)[None]
    psumsq_ref[...] = jnp.sum(out_pre * out_pre, axis=0, keepdims=True)[None]


def _compute_msg(atom, qg3, nbr_fea, idxf3, w1a, w1c, b1r, w2, b2r):
    return pl.pallas_call(
        _msg_kernel,
        grid=(NBLK,),
        in_specs=[
            pl.BlockSpec((BN_BLK, F), lambda i: (i, 0)),
            pl.BlockSpec((BN_BLK, M, H), lambda i: (i, 0, 0)),
            pl.BlockSpec((BN_BLK, M, D_E), lambda i: (i, 0, 0)),
            pl.BlockSpec((BN_BLK, M, 1), lambda i: (i, 0, 0)),
            pl.BlockSpec((F, H), lambda i: (0, 0)),
            pl.BlockSpec((D_E, H), lambda i: (0, 0)),
            pl.BlockSpec((1, H), lambda i: (0, 0)),
            pl.BlockSpec((H, F), lambda i: (0, 0)),
            pl.BlockSpec((1, F), lambda i: (0, 0)),
        ],
        out_specs=[
            pl.BlockSpec((BN_BLK, F), lambda i: (i, 0)),
            pl.BlockSpec((1, 1, F), lambda i: (i, 0, 0)),
            pl.BlockSpec((1, 1, F), lambda i: (i, 0, 0)),
        ],
        out_shape=[
            jax.ShapeDtypeStruct((N, F), jnp.float32),
            jax.ShapeDtypeStruct((NBLK, 1, F), jnp.float32),
            jax.ShapeDtypeStruct((NBLK, 1, F), jnp.float32),
        ],
    )(atom, qg3, nbr_fea, idxf3, w1a, w1c, b1r, w2, b2r)


def _bn_kernel(x_ref, psum_ref, psumsq_ref, gamma_ref, beta_ref, out_ref):
    mean = jnp.sum(psum_ref[...], axis=0) / N
    ex2 = jnp.sum(psumsq_ref[...], axis=0) / N
    var = ex2 - mean * mean
    inv = lax.rsqrt(var + 1e-5)
    y = (x_ref[...] - mean) * (inv * gamma_ref[...]) + beta_ref[...]
    out_ref[...] = _softplus(y)


def _apply_bn(x, psum, psumsq, gammar, betar):
    return pl.pallas_call(
        _bn_kernel,
        grid=(NBLK,),
        in_specs=[
            pl.BlockSpec((BN_BLK, F), lambda i: (i, 0)),
            pl.BlockSpec((NBLK, 1, F), lambda i: (0, 0, 0)),
            pl.BlockSpec((NBLK, 1, F), lambda i: (0, 0, 0)),
            pl.BlockSpec((1, F), lambda i: (0, 0)),
            pl.BlockSpec((1, F), lambda i: (0, 0)),
        ],
        out_specs=pl.BlockSpec((BN_BLK, F), lambda i: (i, 0)),
        out_shape=jax.ShapeDtypeStruct((N, F), jnp.float32),
    )(x, psum, psumsq, gammar, betar)


def kernel(atom_in_fea, nbr_fea, nbr_fea_idx, W1, b1, W2, b2,
           bn_gamma, bn_beta):
    w1a = W1[:F]
    w1b = W1[F:2 * F]
    w1c = W1[2 * F:]
    b1r = b1.reshape(1, H)
    b2r = b2.reshape(1, F)
    gammar = bn_gamma.reshape(1, F)
    betar = bn_beta.reshape(1, F)
    idx_flat = nbr_fea_idx.reshape(N * M)
    idxf3 = nbr_fea_idx.astype(jnp.float32).reshape(N, M, 1)

    q = _compute_q(atom_in_fea, w1b)
    qg = _gather_rows(q, idx_flat).reshape(N, M, H)
    out_pre, psum, psumsq = _compute_msg(
        atom_in_fea, qg, nbr_fea, idxf3, w1a, w1c, b1r, W2, b2r)
    return _apply_bn(out_pre, psum, psumsq, gammar, betar)
